# Initial kernel scaffold; baseline (speedup 1.0000x reference)
import jax
import jax.numpy as jnp
from jax.experimental import pallas as pl


def kernel(input, embed):
    raise NotImplementedError("write your pallas kernel here")


# speedup vs baseline: 2.3049x; 2.3049x over previous
import functools

import jax
import jax.numpy as jnp
from jax import lax
from jax.experimental import pallas as pl
from jax.experimental.pallas import tpu as pltpu
from jax.experimental.pallas import tpu_sc as plsc


def _argmin_body(x_ref, e_ref, lidx_ref, gidx_ref, min_s, arg_s):
    k = pl.program_id(0)
    i = pl.program_id(1)
    nk = pl.num_programs(0)
    H, Kb, hd = e_ref.shape
    Tn = x_ref.shape[2]
    k_total = nk * Kb
    ncol = pl.ds(i * Tn, Tn)
    koff = k * Kb

    @pl.when(k == 0)
    def _init():
        min_s[:, ncol] = jnp.full((H, Tn), jnp.inf, jnp.float32)
        arg_s[:, ncol] = jnp.zeros((H, Tn), jnp.int32)

    for h in range(H):
        eh = e_ref[h]
        xh = x_ref[h]
        xe = lax.dot_general(
            eh.astype(jnp.bfloat16), xh.astype(jnp.bfloat16),
            (((1,), (0,)), ((), ())),
            preferred_element_type=jnp.float32,
        )
        e2 = jnp.sum(eh * eh, axis=1, keepdims=True)
        x2 = jnp.sum(xh * xh, axis=0, keepdims=True)
        d = (x2 - 2.0 * xe) + e2
        bmin = jnp.min(d, axis=0, keepdims=True)
        iot = lax.broadcasted_iota(jnp.int32, d.shape, 0)
        barg = jnp.min(jnp.where(d <= bmin, iot, Kb), axis=0, keepdims=True)
        row = pl.ds(h, 1)
        cur = min_s[row, ncol]
        curarg = arg_s[row, ncol]
        better = bmin < cur
        min_s[row, ncol] = jnp.where(better, bmin, cur)
        arg_s[row, ncol] = jnp.where(better, barg + koff, curarg)

    @pl.when(k == nk - 1)
    def _finish():
        la = arg_s[:, ncol]
        lidx_ref[...] = la
        offs = lax.broadcasted_iota(jnp.int32, la.shape, 0) * k_total
        gidx_ref[...] = la + offs


def _argmin_call(x_t, et, Tn=512, Kb=2048):
    H, hd, N = x_t.shape
    K = et.shape[1]
    grid = (K // Kb, N // Tn)
    return pl.pallas_call(
        _argmin_body,
        grid=grid,
        in_specs=[
            pl.BlockSpec((H, hd, Tn), lambda k, i: (0, 0, i)),
            pl.BlockSpec((H, Kb, hd), lambda k, i: (0, k, 0)),
        ],
        out_specs=[
            pl.BlockSpec((H, Tn), lambda k, i: (0, i)),
            pl.BlockSpec((H, Tn), lambda k, i: (0, i)),
        ],
        out_shape=[
            jax.ShapeDtypeStruct((H, N), jnp.int32),
            jax.ShapeDtypeStruct((H, N), jnp.int32),
        ],
        scratch_shapes=[
            pltpu.VMEM((H, N), jnp.float32),
            pltpu.VMEM((H, N), jnp.int32),
        ],
    )(x_t, et)


def _make_sc_gather(V, Dd, Bg):
    info = plsc.get_sparse_core_info()
    NC, NS = info.num_cores, info.num_subcores
    NW = NC * NS
    per_w = Bg // NW
    C = 128
    n_ch = per_w // C
    mesh = plsc.VectorSubcoreMesh(core_axis_name="c", subcore_axis_name="s")

    @functools.partial(
        pl.kernel,
        mesh=mesh,
        out_type=jax.ShapeDtypeStruct((Bg, Dd), jnp.float32),
        scratch_types=[
            pltpu.VMEM((per_w,), jnp.int32),
            pltpu.VMEM((2, C, Dd), jnp.float32),
            pltpu.SemaphoreType.DMA,
            pltpu.SemaphoreType.DMA,
        ],
    )
    def gk(table_hbm, idx_hbm, out_hbm, idx_v, rows_v, sem_g, sem_g2):
        wid = lax.axis_index("s") * NC + lax.axis_index("c")
        base = wid * per_w
        pltpu.sync_copy(idx_hbm.at[pl.ds(base, per_w)], idx_v)
        sems = (sem_g, sem_g2)
        pending = pltpu.async_copy(
            table_hbm.at[idx_v.at[pl.ds(0, C)]], rows_v.at[0], sems[0])
        for c in range(n_ch):
            buf = c % 2
            pending.wait()
            if c + 1 < n_ch:
                nbuf = (c + 1) % 2
                pending = pltpu.async_copy(
                    table_hbm.at[idx_v.at[pl.ds((c + 1) * C, C)]],
                    rows_v.at[nbuf], sems[nbuf])
            pltpu.sync_copy(rows_v.at[buf], out_hbm.at[pl.ds(base + c * C, C)])

    return gk


def _diff_body(x_ref, q_ref, o_ref, acc_s):
    i = pl.program_id(0)

    @pl.when(i == 0)
    def _init():
        acc_s[0] = 0.0

    acc_s[0] += jnp.sum(jnp.abs(x_ref[...] - q_ref[...]))

    @pl.when(i == pl.num_programs(0) - 1)
    def _finish():
        o_ref[0] = acc_s[0]


def _diff_call(x, q, Tb=2048):
    N, D = x.shape
    grid = (N // Tb,)
    total = pl.pallas_call(
        _diff_body,
        grid=grid,
        in_specs=[
            pl.BlockSpec((Tb, D), lambda i: (i, 0)),
            pl.BlockSpec((Tb, D), lambda i: (i, 0)),
        ],
        out_specs=pl.BlockSpec(memory_space=pltpu.SMEM),
        out_shape=jax.ShapeDtypeStruct((1,), jnp.float32),
        scratch_shapes=[pltpu.SMEM((1,), jnp.float32)],
    )(x, q)
    return (total / float(N * D)).reshape(())


def kernel(input, embed):
    b, s, d = input.shape
    H, hd, K = embed.shape
    N = b * s

    x = input.reshape(N, d)
    x_t = x.reshape(N, H, hd).transpose(1, 2, 0)
    et = embed.transpose(0, 2, 1)

    lidx, gidx = _argmin_call(x_t, et)

    table = jnp.concatenate(
        [et, jnp.zeros((H, K, 128 - hd), jnp.float32)], axis=2
    ).reshape(H * K, 128)
    gidx_flat = gidx.T.reshape(N * H)
    q = _make_sc_gather(H * K, 128, N * H)(table, gidx_flat)

    q_nd = q[:, :hd].reshape(N, d)
    diff = _diff_call(x, q_nd)

    return (q_nd.reshape(b, s, d), diff, lidx.T.reshape(b, s, H))
